# baseline (speedup 1.0000x reference)
import functools

import jax
import jax.numpy as jnp
from jax import lax
from jax.experimental import pallas as pl
from jax.experimental.pallas import tpu as pltpu
from jax.experimental.pallas import tpu_sc as plsc

NC = 2
NS = 16
NW = NC * NS


def _deg_body(g4, ones_h, zeros_h, out, sidx, didx, ones_v, zbuf,
              dbuf, acc_s, acc_d, ss, sd):
    npad = acc_s.shape[0]
    per_tile = npad // NS
    cid = lax.axis_index("c")
    sid = lax.axis_index("s")
    wid = cid * NS + sid
    nch = sidx.shape[0]
    depth = 8

    pltpu.sync_copy(g4.at[0, wid], sidx)
    pltpu.sync_copy(g4.at[1, wid], didx)
    pltpu.sync_copy(ones_h, ones_v)
    pltpu.sync_copy(zeros_h, zbuf)

    base = sid * per_tile
    pltpu.sync_copy(zbuf, acc_s.at[pl.ds(base, per_tile)])
    pltpu.sync_copy(zbuf, acc_d.at[pl.ds(base, per_tile)])
    plsc.subcore_barrier()

    unroll = 5
    ngr = nch // unroll
    assert nch % unroll == 0

    def body(gi, carry):
        for u in range(unroll):
            j = gi * unroll + u
            pltpu.async_copy(ones_v, acc_s.at[sidx.at[j]], ss, add=True)
            pltpu.async_copy(ones_v, acc_d.at[didx.at[j]], sd, add=True)

        @pl.when(gi >= depth)
        def _():
            for _u in range(unroll):
                pltpu.make_async_copy(ones_v, acc_s.at[sidx.at[0]],
                                      ss).wait()
                pltpu.make_async_copy(ones_v, acc_d.at[didx.at[0]],
                                      sd).wait()

        return carry

    lax.fori_loop(0, ngr, body, 0)

    def drain(gi, carry):
        for _u in range(unroll):
            pltpu.make_async_copy(ones_v, acc_s.at[sidx.at[0]], ss).wait()
            pltpu.make_async_copy(ones_v, acc_d.at[didx.at[0]], sd).wait()
        return carry

    lax.fori_loop(0, min(depth, ngr), drain, 0)
    plsc.subcore_barrier()

    pltpu.sync_copy(acc_s.at[pl.ds(base, per_tile)], dbuf)
    pltpu.sync_copy(dbuf, out.at[pl.ds(cid * 2 * npad + base, per_tile)])
    pltpu.sync_copy(acc_d.at[pl.ds(base, per_tile)], dbuf)
    pltpu.sync_copy(dbuf, out.at[pl.ds((cid * 2 + 1) * npad + base,
                                       per_tile)])


NBUF = 4


def _agg_body(ys, g4, zeros_h, out,
              sb0, sb1, sb2, sb3, db0, db1, db2, db3, r0, r1, r2, r3, acc,
              ss0, ss1, ss2, ss3, ds0, ds1, ds2, ds3,
              gs0, gs1, gs2, gs3, cs0, cs1, cs2, cs3):
    npad, dd = acc.shape
    rows_per_tile = npad // NS
    chunk = r0.shape[0]
    nch = g4.shape[2]
    ncopy = rows_per_tile // chunk
    cid = lax.axis_index("c")
    sid = lax.axis_index("s")
    wid = cid * NS + sid

    sb = (sb0, sb1, sb2, sb3)
    db = (db0, db1, db2, db3)
    rows = (r0, r1, r2, r3)
    ssem = (ss0, ss1, ss2, ss3)
    dsem = (ds0, ds1, ds2, ds3)
    gsem = (gs0, gs1, gs2, gs3)
    csem = (cs0, cs1, cs2, cs3)

    pltpu.sync_copy(zeros_h, r0)
    rowbase = sid * rows_per_tile
    for k in range(ncopy):
        sl = pl.ds(rowbase + k * chunk, chunk)
        pltpu.async_copy(r0, acc.at[sl], gs0)
    for k in range(ncopy):
        sl = pl.ds(rowbase + k * chunk, chunk)
        pltpu.make_async_copy(r0, acc.at[sl], gs0).wait()
    plsc.subcore_barrier()

    def _prep(t, j):
        pltpu.async_copy(g4.at[0, wid, j], sb[t], ssem[t])
        pltpu.async_copy(g4.at[1, wid, j], db[t], dsem[t])
        pltpu.make_async_copy(g4.at[0, wid, 0], sb[t], ssem[t]).wait()
        pltpu.async_copy(ys.at[sb[t]], rows[t], gsem[t])

    for b in range(min(3, nch)):
        _prep(b, b)

    def group(gi, carry):
        for b in range(NBUF):
            j = NBUF * gi + b
            s = b
            pltpu.make_async_copy(ys.at[pl.ds(0, chunk)], rows[s],
                                  gsem[s]).wait()
            pltpu.make_async_copy(g4.at[1, wid, 0], db[s], dsem[s]).wait()
            pltpu.async_copy(rows[s], acc.at[db[s]], csem[s], add=True)
            t = (b + 3) % NBUF

            @pl.when(j + 3 < nch)
            def _():
                @pl.when(j >= 1)
                def _():
                    pltpu.make_async_copy(rows[t], acc.at[db[t]],
                                          csem[t]).wait()

                _prep(t, j + 3)

        return carry

    ngroups = (nch - 1) // NBUF
    lax.fori_loop(0, ngroups, group, 0)
    for j in range(ngroups * NBUF, nch):
        s = j % NBUF
        pltpu.make_async_copy(ys.at[pl.ds(0, chunk)], rows[s],
                              gsem[s]).wait()
        pltpu.make_async_copy(g4.at[1, wid, 0], db[s], dsem[s]).wait()
        pltpu.async_copy(rows[s], acc.at[db[s]], csem[s], add=True)
    for j in range(max(nch - NBUF, 0), nch):
        s = j % NBUF
        pltpu.make_async_copy(rows[s], acc.at[db[s]], csem[s]).wait()
    plsc.subcore_barrier()

    for k in range(ncopy):
        b = k % 2
        sl = pl.ds(rowbase + k * chunk, chunk)
        if k >= 2:
            slp = pl.ds(rowbase + (k - 2) * chunk, chunk)
            pltpu.make_async_copy(rows[b], out.at[cid, slp],
                                  gsem[b]).wait()
        pltpu.sync_copy(acc.at[sl], rows[b])
        pltpu.async_copy(rows[b], out.at[cid, sl], gsem[b])
    for k in range(max(ncopy - 2, 0), ncopy):
        b = k % 2
        sl = pl.ds(rowbase + k * chunk, chunk)
        pltpu.make_async_copy(rows[b], out.at[cid, sl], gsem[b]).wait()


def _mm_body(x_ref, w_ref, o_ref):
    o_ref[...] = jnp.dot(x_ref[...], w_ref[...],
                         preferred_element_type=jnp.float32)


def _scale_body(z_ref, deg_ref, o_ref):
    norm = lax.rsqrt(jnp.maximum(deg_ref[:, 0:1], 1.0))
    o_ref[...] = z_ref[...] * norm


def _fin_body(a0_ref, a1_ref, deg_ref, b_ref, o_ref):
    norm = lax.rsqrt(jnp.maximum(deg_ref[:, 1:2], 1.0))
    s = (a0_ref[0] + a1_ref[0]) * norm + b_ref[...]
    o_ref[...] = jnp.maximum(s, 0.0)


def kernel(g, features, W, b):
    n, d = features.shape
    e = g.shape[1]
    d_out = W.shape[1]

    chunk = 80
    assert e % (NW * chunk) == 0
    nch = e // (NW * chunk)
    npad = ((n + NS * 128 - 1) // (NS * 128)) * (NS * 128)
    deg_pt = npad // NS
    assert (npad // NS) % chunk == 0

    g32 = g.astype(jnp.int32)
    g4 = g32.reshape(2, NW, nch, chunk)

    mesh = plsc.VectorSubcoreMesh(core_axis_name="c", subcore_axis_name="s",
                                  num_cores=NC, num_subcores=NS)

    deg_fn = functools.partial(
        pl.kernel,
        out_type=jax.ShapeDtypeStruct((NC * 2 * npad,), jnp.float32),
        mesh=mesh,
        scratch_types=[
            pltpu.VMEM((nch, chunk), jnp.int32),
            pltpu.VMEM((nch, chunk), jnp.int32),
            pltpu.VMEM((chunk,), jnp.float32),
            pltpu.VMEM((deg_pt,), jnp.float32),
            pltpu.VMEM((deg_pt,), jnp.float32),
            pltpu.VMEM_SHARED((npad,), jnp.float32),
            pltpu.VMEM_SHARED((npad,), jnp.float32),
            pltpu.SemaphoreType.DMA,
            pltpu.SemaphoreType.DMA,
        ],
    )(_deg_body)
    degp = deg_fn(g4,
                  jnp.ones((chunk,), jnp.float32),
                  jnp.zeros((deg_pt,), jnp.float32))
    deg2t = degp.reshape(2, 2, npad).sum(axis=0).T

    blk = 2000
    grid = (n // blk,)
    z = pl.pallas_call(
        _mm_body,
        grid=grid,
        in_specs=[
            pl.BlockSpec((blk, d), lambda i: (i, 0)),
            pl.BlockSpec((d, d_out), lambda i: (0, 0)),
        ],
        out_specs=pl.BlockSpec((blk, d_out), lambda i: (i, 0)),
        out_shape=jax.ShapeDtypeStruct((n, d_out), jnp.float32),
    )(features, W)

    ys = pl.pallas_call(
        _scale_body,
        grid=grid,
        in_specs=[
            pl.BlockSpec((blk, d_out), lambda i: (i, 0)),
            pl.BlockSpec((blk, 2), lambda i: (i, 0)),
        ],
        out_specs=pl.BlockSpec((blk, d_out), lambda i: (i, 0)),
        out_shape=jax.ShapeDtypeStruct((n, d_out), jnp.float32),
    )(z, deg2t)

    agg_fn = functools.partial(
        pl.kernel,
        out_type=jax.ShapeDtypeStruct((NC, npad, d_out), jnp.float32),
        mesh=mesh,
        scratch_types=(
            [pltpu.VMEM((chunk,), jnp.int32) for _ in range(2 * NBUF)]
            + [pltpu.VMEM((chunk, d_out), jnp.float32) for _ in range(NBUF)]
            + [pltpu.VMEM_SHARED((npad, d_out), jnp.float32)]
            + [pltpu.SemaphoreType.DMA for _ in range(4 * NBUF)]
        ),
    )(_agg_body)
    parts = agg_fn(ys, g4,
                   jnp.zeros((chunk, d_out), jnp.float32))

    out = pl.pallas_call(
        _fin_body,
        grid=grid,
        in_specs=[
            pl.BlockSpec((1, blk, d_out), lambda i: (0, i, 0)),
            pl.BlockSpec((1, blk, d_out), lambda i: (1, i, 0)),
            pl.BlockSpec((blk, 2), lambda i: (i, 0)),
            pl.BlockSpec((1, d_out), lambda i: (0, 0)),
        ],
        out_specs=pl.BlockSpec((blk, d_out), lambda i: (i, 0)),
        out_shape=jax.ShapeDtypeStruct((n, d_out), jnp.float32),
    )(parts, parts, deg2t, b.reshape(1, d_out))

    return (g, out)
